# Initial kernel scaffold; baseline (speedup 1.0000x reference)
import jax
import jax.numpy as jnp
from jax.experimental import pallas as pl


def kernel(x, A_in, A_motif, coords, W_gat, att_src, att_dst, b_gat, W_gcn, b_gcn, bnA_g, bnA_b, bnM_g, bnM_b, mu, tau, W_prune, b_prune, W_rewire, b_rewire, W_pool, b_pool):
    raise NotImplementedError("write your pallas kernel here")


# speedup vs baseline: 7.1248x; 7.1248x over previous
import jax
import jax.numpy as jnp
from jax.experimental import pallas as pl

N = 2048
IN_C = 128
H = 64
HEADS = 2
D = HEADS * H
K_POOL = 64
MOTIF_TOPK = 8
RADIUS = 2.0
TOPK_ADD = 2

R = 256
GRID = N // R
NEG_INF = float("-inf")
F32 = jnp.float32


def _sigmoid(v):
    return 1.0 / (1.0 + jnp.exp(-v))


def _pre_kernel(x_ref, wgat_ref, wgcn_ref, ast_ref, adt_ref,
                hx_ref, as_ref, ad_ref, xg_ref):
    x = x_ref[...]
    hx = jnp.dot(x, wgat_ref[...], preferred_element_type=F32)
    hx_ref[...] = hx
    xg_ref[...] = jnp.dot(x, wgcn_ref[...], preferred_element_type=F32)
    a_s = []
    a_d = []
    for hd in range(HEADS):
        hxh = hx[:, hd * H:(hd + 1) * H]
        a_s.append(jnp.dot(hxh, ast_ref[:, hd:hd + 1], preferred_element_type=F32))
        a_d.append(jnp.dot(hxh, adt_ref[:, hd:hd + 1], preferred_element_type=F32))
    as_ref[...] = jnp.concatenate(a_s, axis=1)
    ad_ref[...] = jnp.concatenate(a_d, axis=1)


def _gat_kth_kernel(ain_ref, am_ref, hx_ref, ast_ref, ad_ref, bgat_ref,
                    outa_ref, kth_ref):
    i = pl.program_id(0)
    ain = ain_ref[...]
    rows = jax.lax.broadcasted_iota(jnp.int32, (R, N), 0) + i * R
    cols = jax.lax.broadcasted_iota(jnp.int32, (R, N), 1)
    mask = (ain > 0) | (rows == cols)
    outs = []
    for hd in range(HEADS):
        e = ad_ref[:, hd:hd + 1] + ast_ref[hd:hd + 1, :]
        e = jnp.where(e >= 0, e, 0.2 * e)
        e = jnp.where(mask, e, NEG_INF)
        m = jnp.max(e, axis=1, keepdims=True)
        p = jnp.where(mask, jnp.exp(e - m), 0.0)
        s = jnp.sum(p, axis=1, keepdims=True)
        outs.append(jnp.dot(p / s, hx_ref[:, hd * H:(hd + 1) * H],
                            preferred_element_type=F32))
    outa_ref[...] = jnp.concatenate(outs, axis=1) + bgat_ref[...]
    work = am_ref[...]
    m8 = jnp.max(work, axis=1, keepdims=True)
    for _ in range(MOTIF_TOPK - 1):
        work = jnp.where(work >= m8, NEG_INF, work)
        m8 = jnp.max(work, axis=1, keepdims=True)
    kth_ref[...] = m8


def _mhat_kernel(am_ref, kthc_ref, kthr_ref, mhat_ref, dinv_ref):
    a = am_ref[...]
    keep = ((a >= kthc_ref[...]) | (a >= kthr_ref[...])) & (a > 0)
    m = jnp.where(keep, a, 0.0)
    mhat_ref[...] = m
    deg = jnp.sum(m, axis=1, keepdims=True)
    dinv_ref[...] = jnp.where(deg > 0,
                              jax.lax.rsqrt(jnp.where(deg > 0, deg, 1.0)), 0.0)


def _gcn_kernel(mhat_ref, dinvc_ref, dinvr_ref, xg_ref, bgcn_ref, outm_ref):
    t = mhat_ref[...] * dinvr_ref[...]
    outm_ref[...] = (dinvc_ref[...] *
                     jnp.dot(t, xg_ref[...], preferred_element_type=F32) +
                     bgcn_ref[...])


def _bn_kernel(outa_ref, outm_ref, bnag_ref, bnab_ref, bnmg_ref, bnmb_ref,
               mu_ref, wpool_ref, wg4_ref, h_ref, hp_ref, gv_ref):
    def bn(v, g, b):
        m = jnp.mean(v, axis=0, keepdims=True)
        var = jnp.mean((v - m) ** 2, axis=0, keepdims=True)
        return (v - m) * jax.lax.rsqrt(var + 1e-5) * g + b

    def elu(v):
        return jnp.where(v > 0, v, jnp.exp(jnp.minimum(v, 0.0)) - 1.0)

    h_a = elu(bn(outa_ref[...], bnag_ref[...], bnab_ref[...]))
    h_m = elu(bn(outm_ref[...], bnmg_ref[...], bnmb_ref[...]))
    mu = mu_ref[...]
    softplus_mu = jnp.maximum(mu, 0.0) + jnp.log1p(jnp.exp(-jnp.abs(mu)))
    h = h_a + softplus_mu * h_m
    h_ref[...] = h
    hp_ref[...] = jnp.dot(h, wpool_ref[...], preferred_element_type=F32)
    gv_ref[...] = jnp.dot(h, wg4_ref[...], preferred_element_type=F32)


def _pair_logits(upper, gvc, gvr, ca, cb, mh, wm, bias):
    l = jnp.where(upper, gvc[:, ca:ca + 1] + gvr[cb:cb + 1, :],
                  gvr[ca:ca + 1, :] + gvc[:, cb:cb + 1])
    return l + mh * wm + bias


def _gate(l, tau_c):
    return jnp.clip(_sigmoid(l / tau_c) * 1.2 - 0.1, 0.0, 1.0)


def _kth2_kernel(ain_ref, mhat_ref, gv_ref, gvt_ref, crd_ref, crdt_ref,
                 wmrw_ref, brw_ref, tau_ref, kth2_ref):
    i = pl.program_id(0)
    rows = jax.lax.broadcasted_iota(jnp.int32, (R, N), 0) + i * R
    cols = jax.lax.broadcasted_iota(jnp.int32, (R, N), 1)
    upper = cols > rows
    dist = (jnp.abs(crd_ref[:, 0:1] - crdt_ref[0:1, :]) +
            jnp.abs(crd_ref[:, 1:2] - crdt_ref[1:2, :]))
    cand = (dist > 0) & (dist <= RADIUS) & (ain_ref[...] < 1e-6)
    tau_c = jnp.maximum(tau_ref[...], 0.1)
    l = _pair_logits(upper, gv_ref[...], gvt_ref[...], 2, 3,
                     mhat_ref[...], wmrw_ref[...], brw_ref[...])
    z = _gate(l, tau_c)
    scores = jnp.where(cand, z, NEG_INF)
    m1 = jnp.max(scores, axis=1, keepdims=True)
    c = jnp.sum(jnp.where(scores == m1, 1.0, 0.0), axis=1, keepdims=True)
    m2 = jnp.max(jnp.where(scores < m1, scores, NEG_INF), axis=1, keepdims=True)
    kth2_ref[...] = jnp.where(c >= 2.0, m1, m2)


def _anew_kernel(ain_ref, mhat_ref, gv_ref, gvt_ref, crd_ref, crdt_ref,
                 wmpr_ref, bpr_ref, wmrw_ref, brw_ref, tau_ref,
                 k2c_ref, k2r_ref, anew_ref, dpi_ref):
    i = pl.program_id(0)
    rows = jax.lax.broadcasted_iota(jnp.int32, (R, N), 0) + i * R
    cols = jax.lax.broadcasted_iota(jnp.int32, (R, N), 1)
    upper = cols > rows
    ain = ain_ref[...]
    mh = mhat_ref[...]
    tau_c = jnp.maximum(tau_ref[...], 0.1)
    l_pr = _pair_logits(upper, gv_ref[...], gvt_ref[...], 0, 1,
                        mh, wmpr_ref[...], bpr_ref[...])
    a_pruned = ain * _gate(l_pr, tau_c)
    dist = (jnp.abs(crd_ref[:, 0:1] - crdt_ref[0:1, :]) +
            jnp.abs(crd_ref[:, 1:2] - crdt_ref[1:2, :]))
    cand = (dist > 0) & (dist <= RADIUS) & (ain < 1e-6)
    l_rw = _pair_logits(upper, gv_ref[...], gvt_ref[...], 2, 3,
                        mh, wmrw_ref[...], brw_ref[...])
    z = _gate(l_rw, tau_c)
    add = cand & ((z >= k2c_ref[...]) | (z >= k2r_ref[...]))
    anew = a_pruned + jnp.where(add, z, 0.0)
    anew_ref[...] = anew
    dp = jnp.sum(anew, axis=1, keepdims=True) + 1.0
    dpi_ref[...] = jax.lax.rsqrt(dp)


def _pool_kernel(anew_ref, dpic_ref, dpir_ref, hp_ref, bpool_ref, s_ref):
    i = pl.program_id(0)
    hp_blk = hp_ref[pl.ds(i * R, R), :]
    dpic = dpic_ref[...]
    t = jnp.dot(anew_ref[...] * dpir_ref[...], hp_ref[...],
                preferred_element_type=F32)
    logits = dpic * (t + dpic * hp_blk) + bpool_ref[...]
    m = jnp.max(logits, axis=1, keepdims=True)
    e = jnp.exp(logits - m)
    s_ref[...] = e / jnp.sum(e, axis=1, keepdims=True)


def _final_kernel(anew_ref, s_ref, h_ref, apool_ref, xpool_ref):
    i = pl.program_id(0)
    s_blk = s_ref[pl.ds(i * R, R), :]
    h_blk = h_ref[pl.ds(i * R, R), :]
    y = jnp.dot(anew_ref[...], s_ref[...], preferred_element_type=F32)
    dn = (((0,), (0,)), ((), ()))
    ca = jax.lax.dot_general(s_blk, y, dn, preferred_element_type=F32)
    cx = jax.lax.dot_general(s_blk, h_blk, dn, preferred_element_type=F32)

    @pl.when(i == 0)
    def _():
        apool_ref[...] = ca
        xpool_ref[...] = cx

    @pl.when(i > 0)
    def _():
        apool_ref[...] = apool_ref[...] + ca
        xpool_ref[...] = xpool_ref[...] + cx


def _blk(shape):
    return pl.BlockSpec(shape, lambda i: (i, 0))


def _full(shape):
    return pl.BlockSpec(shape, lambda i: (0, 0))


def _sds(*shape):
    return jax.ShapeDtypeStruct(shape, F32)


def kernel(x, A_in, A_motif, coords, W_gat, att_src, att_dst, b_gat,
           W_gcn, b_gcn, bnA_g, bnA_b, bnM_g, bnM_b, mu, tau,
           W_prune, b_prune, W_rewire, b_rewire, W_pool, b_pool):
    astT = att_src.T
    adtT = att_dst.T
    bgat = b_gat.reshape(1, D)
    bgcn = b_gcn.reshape(1, D)
    bpool = b_pool.reshape(1, K_POOL)
    wg4 = jnp.stack([W_prune[:D], W_prune[D:2 * D],
                     W_rewire[:D], W_rewire[D:2 * D]], axis=1)
    wmpr = W_prune[2 * D].reshape(1, 1)
    bpr = jnp.asarray(b_prune, dtype=F32).reshape(1, 1)
    wmrw = W_rewire[2 * D].reshape(1, 1)
    brw = jnp.asarray(b_rewire, dtype=F32).reshape(1, 1)
    tau2 = jnp.asarray(tau, dtype=F32).reshape(1, 1)
    mu2 = jnp.asarray(mu, dtype=F32).reshape(1, 1)

    hx, a_s, a_d, xg = pl.pallas_call(
        _pre_kernel,
        out_shape=[_sds(N, D), _sds(N, HEADS), _sds(N, HEADS), _sds(N, D)],
    )(x, W_gat, W_gcn, astT, adtT)
    asT = a_s.T

    out_A, kth = pl.pallas_call(
        _gat_kth_kernel,
        grid=(GRID,),
        in_specs=[_blk((R, N)), _blk((R, N)), _full((N, D)),
                  _full((HEADS, N)), _blk((R, HEADS)), _full((1, D))],
        out_specs=[_blk((R, D)), _blk((R, 1))],
        out_shape=[_sds(N, D), _sds(N, 1)],
    )(A_in, A_motif, hx, asT, a_d, bgat)

    M_hat, dinv = pl.pallas_call(
        _mhat_kernel,
        grid=(GRID,),
        in_specs=[_blk((R, N)), _blk((R, 1)), _full((1, N))],
        out_specs=[_blk((R, N)), _blk((R, 1))],
        out_shape=[_sds(N, N), _sds(N, 1)],
    )(A_motif, kth, kth.reshape(1, N))

    out_M = pl.pallas_call(
        _gcn_kernel,
        grid=(GRID,),
        in_specs=[_blk((R, N)), _blk((R, 1)), _full((1, N)),
                  _full((N, D)), _full((1, D))],
        out_specs=_blk((R, D)),
        out_shape=_sds(N, D),
    )(M_hat, dinv, dinv.reshape(1, N), xg, bgcn)

    h, hp, gv = pl.pallas_call(
        _bn_kernel,
        out_shape=[_sds(N, D), _sds(N, K_POOL), _sds(N, 4)],
    )(out_A, out_M, bnA_g.reshape(1, D), bnA_b.reshape(1, D),
      bnM_g.reshape(1, D), bnM_b.reshape(1, D), mu2, W_pool, wg4)
    gvT = gv.T
    crdT = coords.T

    kth2 = pl.pallas_call(
        _kth2_kernel,
        grid=(GRID,),
        in_specs=[_blk((R, N)), _blk((R, N)), _blk((R, 4)), _full((4, N)),
                  _blk((R, 2)), _full((2, N)),
                  _full((1, 1)), _full((1, 1)), _full((1, 1))],
        out_specs=_blk((R, 1)),
        out_shape=_sds(N, 1),
    )(A_in, M_hat, gv, gvT, coords, crdT, wmrw, brw, tau2)

    A_new, dpi = pl.pallas_call(
        _anew_kernel,
        grid=(GRID,),
        in_specs=[_blk((R, N)), _blk((R, N)), _blk((R, 4)), _full((4, N)),
                  _blk((R, 2)), _full((2, N)),
                  _full((1, 1)), _full((1, 1)), _full((1, 1)), _full((1, 1)),
                  _full((1, 1)), _blk((R, 1)), _full((1, N))],
        out_specs=[_blk((R, N)), _blk((R, 1))],
        out_shape=[_sds(N, N), _sds(N, 1)],
    )(A_in, M_hat, gv, gvT, coords, crdT, wmpr, bpr, wmrw, brw, tau2,
      kth2, kth2.reshape(1, N))

    S = pl.pallas_call(
        _pool_kernel,
        grid=(GRID,),
        in_specs=[_blk((R, N)), _blk((R, 1)), _full((1, N)),
                  _full((N, K_POOL)), _full((1, K_POOL))],
        out_specs=_blk((R, K_POOL)),
        out_shape=_sds(N, K_POOL),
    )(A_new, dpi, dpi.reshape(1, N), hp, bpool)

    A_pool, x_pool = pl.pallas_call(
        _final_kernel,
        grid=(GRID,),
        in_specs=[_blk((R, N)), _full((N, K_POOL)), _full((N, D))],
        out_specs=[_full((K_POOL, K_POOL)), _full((K_POOL, D))],
        out_shape=[_sds(K_POOL, K_POOL), _sds(K_POOL, D)],
    )(A_new, S, h)

    return x_pool, A_pool, S


# speedup vs baseline: 7.1741x; 1.0069x over previous
import jax
import jax.numpy as jnp
from jax.experimental import pallas as pl

N = 2048
IN_C = 128
H = 64
HEADS = 2
D = HEADS * H
K_POOL = 64
MOTIF_TOPK = 8
RADIUS = 2.0
TOPK_ADD = 2

R = 512
GRID = N // R
NEG_INF = float("-inf")
F32 = jnp.float32


def _sigmoid(v):
    return 1.0 / (1.0 + jnp.exp(-v))


def _pre_kernel(x_ref, wgat_ref, wgcn_ref, ast_ref, adt_ref,
                hx_ref, as_ref, ad_ref, xg_ref):
    x = x_ref[...]
    hx = jnp.dot(x, wgat_ref[...], preferred_element_type=F32)
    hx_ref[...] = hx
    xg_ref[...] = jnp.dot(x, wgcn_ref[...], preferred_element_type=F32)
    a_s = []
    a_d = []
    for hd in range(HEADS):
        hxh = hx[:, hd * H:(hd + 1) * H]
        a_s.append(jnp.dot(hxh, ast_ref[:, hd:hd + 1], preferred_element_type=F32))
        a_d.append(jnp.dot(hxh, adt_ref[:, hd:hd + 1], preferred_element_type=F32))
    as_ref[...] = jnp.concatenate(a_s, axis=1)
    ad_ref[...] = jnp.concatenate(a_d, axis=1)


def _gat_kth_kernel(ain_ref, am_ref, hx_ref, ast_ref, ad_ref, bgat_ref,
                    outa_ref, kth_ref):
    i = pl.program_id(0)
    ain = ain_ref[...]
    rows = jax.lax.broadcasted_iota(jnp.int32, (R, N), 0) + i * R
    cols = jax.lax.broadcasted_iota(jnp.int32, (R, N), 1)
    mask = (ain > 0) | (rows == cols)
    outs = []
    for hd in range(HEADS):
        e = ad_ref[:, hd:hd + 1] + ast_ref[hd:hd + 1, :]
        e = jnp.where(e >= 0, e, 0.2 * e)
        e = jnp.where(mask, e, NEG_INF)
        m = jnp.max(e, axis=1, keepdims=True)
        p = jnp.where(mask, jnp.exp(e - m), 0.0)
        s = jnp.sum(p, axis=1, keepdims=True)
        outs.append(jnp.dot(p / s, hx_ref[:, hd * H:(hd + 1) * H],
                            preferred_element_type=F32))
    outa_ref[...] = jnp.concatenate(outs, axis=1) + bgat_ref[...]
    work = am_ref[...]
    m8 = jnp.max(work, axis=1, keepdims=True)
    for _ in range(MOTIF_TOPK - 1):
        work = jnp.where(work >= m8, NEG_INF, work)
        m8 = jnp.max(work, axis=1, keepdims=True)
    kth_ref[...] = m8


def _mhat_kernel(am_ref, kthc_ref, kthr_ref, mhat_ref, dinv_ref):
    a = am_ref[...]
    keep = ((a >= kthc_ref[...]) | (a >= kthr_ref[...])) & (a > 0)
    m = jnp.where(keep, a, 0.0)
    mhat_ref[...] = m
    deg = jnp.sum(m, axis=1, keepdims=True)
    dinv_ref[...] = jnp.where(deg > 0,
                              jax.lax.rsqrt(jnp.where(deg > 0, deg, 1.0)), 0.0)


def _gcn_kernel(mhat_ref, dinvc_ref, dinvr_ref, xg_ref, bgcn_ref, outm_ref):
    t = mhat_ref[...] * dinvr_ref[...]
    outm_ref[...] = (dinvc_ref[...] *
                     jnp.dot(t, xg_ref[...], preferred_element_type=F32) +
                     bgcn_ref[...])


def _bn_kernel(outa_ref, outm_ref, bnag_ref, bnab_ref, bnmg_ref, bnmb_ref,
               mu_ref, wpool_ref, wg4_ref, h_ref, hp_ref, gv_ref):
    def bn(v, g, b):
        m = jnp.mean(v, axis=0, keepdims=True)
        var = jnp.mean((v - m) ** 2, axis=0, keepdims=True)
        return (v - m) * jax.lax.rsqrt(var + 1e-5) * g + b

    def elu(v):
        return jnp.where(v > 0, v, jnp.exp(jnp.minimum(v, 0.0)) - 1.0)

    h_a = elu(bn(outa_ref[...], bnag_ref[...], bnab_ref[...]))
    h_m = elu(bn(outm_ref[...], bnmg_ref[...], bnmb_ref[...]))
    mu = mu_ref[...]
    softplus_mu = jnp.maximum(mu, 0.0) + jnp.log1p(jnp.exp(-jnp.abs(mu)))
    h = h_a + softplus_mu * h_m
    h_ref[...] = h
    hp_ref[...] = jnp.dot(h, wpool_ref[...], preferred_element_type=F32)
    gv_ref[...] = jnp.dot(h, wg4_ref[...], preferred_element_type=F32)


def _pair_logits(upper, gvc, gvr, ca, cb, mh, wm, bias):
    l = jnp.where(upper, gvc[:, ca:ca + 1] + gvr[cb:cb + 1, :],
                  gvr[ca:ca + 1, :] + gvc[:, cb:cb + 1])
    return l + mh * wm + bias


def _gate(l, tau_c):
    return jnp.clip(_sigmoid(l / tau_c) * 1.2 - 0.1, 0.0, 1.0)


def _kth2_kernel(ain_ref, mhat_ref, gv_ref, gvt_ref, crd_ref, crdt_ref,
                 wmrw_ref, brw_ref, tau_ref, kth2_ref):
    i = pl.program_id(0)
    rows = jax.lax.broadcasted_iota(jnp.int32, (R, N), 0) + i * R
    cols = jax.lax.broadcasted_iota(jnp.int32, (R, N), 1)
    upper = cols > rows
    dist = (jnp.abs(crd_ref[:, 0:1] - crdt_ref[0:1, :]) +
            jnp.abs(crd_ref[:, 1:2] - crdt_ref[1:2, :]))
    cand = (dist > 0) & (dist <= RADIUS) & (ain_ref[...] < 1e-6)
    tau_c = jnp.maximum(tau_ref[...], 0.1)
    l = _pair_logits(upper, gv_ref[...], gvt_ref[...], 2, 3,
                     mhat_ref[...], wmrw_ref[...], brw_ref[...])
    z = _gate(l, tau_c)
    scores = jnp.where(cand, z, NEG_INF)
    m1 = jnp.max(scores, axis=1, keepdims=True)
    c = jnp.sum(jnp.where(scores == m1, 1.0, 0.0), axis=1, keepdims=True)
    m2 = jnp.max(jnp.where(scores < m1, scores, NEG_INF), axis=1, keepdims=True)
    kth2_ref[...] = jnp.where(c >= 2.0, m1, m2)


def _anew_kernel(ain_ref, mhat_ref, gv_ref, gvt_ref, crd_ref, crdt_ref,
                 wmpr_ref, bpr_ref, wmrw_ref, brw_ref, tau_ref,
                 k2c_ref, k2r_ref, anew_ref, dpi_ref):
    i = pl.program_id(0)
    rows = jax.lax.broadcasted_iota(jnp.int32, (R, N), 0) + i * R
    cols = jax.lax.broadcasted_iota(jnp.int32, (R, N), 1)
    upper = cols > rows
    ain = ain_ref[...]
    mh = mhat_ref[...]
    tau_c = jnp.maximum(tau_ref[...], 0.1)
    l_pr = _pair_logits(upper, gv_ref[...], gvt_ref[...], 0, 1,
                        mh, wmpr_ref[...], bpr_ref[...])
    a_pruned = ain * _gate(l_pr, tau_c)
    dist = (jnp.abs(crd_ref[:, 0:1] - crdt_ref[0:1, :]) +
            jnp.abs(crd_ref[:, 1:2] - crdt_ref[1:2, :]))
    cand = (dist > 0) & (dist <= RADIUS) & (ain < 1e-6)
    l_rw = _pair_logits(upper, gv_ref[...], gvt_ref[...], 2, 3,
                        mh, wmrw_ref[...], brw_ref[...])
    z = _gate(l_rw, tau_c)
    add = cand & ((z >= k2c_ref[...]) | (z >= k2r_ref[...]))
    anew = a_pruned + jnp.where(add, z, 0.0)
    anew_ref[...] = anew
    dp = jnp.sum(anew, axis=1, keepdims=True) + 1.0
    dpi_ref[...] = jax.lax.rsqrt(dp)


def _pool_kernel(anew_ref, dpic_ref, dpir_ref, hp_ref, bpool_ref, s_ref):
    i = pl.program_id(0)
    hp_blk = hp_ref[pl.ds(i * R, R), :]
    dpic = dpic_ref[...]
    t = jnp.dot(anew_ref[...] * dpir_ref[...], hp_ref[...],
                preferred_element_type=F32)
    logits = dpic * (t + dpic * hp_blk) + bpool_ref[...]
    m = jnp.max(logits, axis=1, keepdims=True)
    e = jnp.exp(logits - m)
    s_ref[...] = e / jnp.sum(e, axis=1, keepdims=True)


def _final_kernel(anew_ref, s_ref, h_ref, apool_ref, xpool_ref):
    i = pl.program_id(0)
    s_blk = s_ref[pl.ds(i * R, R), :]
    h_blk = h_ref[pl.ds(i * R, R), :]
    y = jnp.dot(anew_ref[...], s_ref[...], preferred_element_type=F32)
    dn = (((0,), (0,)), ((), ()))
    ca = jax.lax.dot_general(s_blk, y, dn, preferred_element_type=F32)
    cx = jax.lax.dot_general(s_blk, h_blk, dn, preferred_element_type=F32)

    @pl.when(i == 0)
    def _():
        apool_ref[...] = ca
        xpool_ref[...] = cx

    @pl.when(i > 0)
    def _():
        apool_ref[...] = apool_ref[...] + ca
        xpool_ref[...] = xpool_ref[...] + cx


def _blk(shape):
    return pl.BlockSpec(shape, lambda i: (i, 0))


def _full(shape):
    return pl.BlockSpec(shape, lambda i: (0, 0))


def _sds(*shape):
    return jax.ShapeDtypeStruct(shape, F32)


def kernel(x, A_in, A_motif, coords, W_gat, att_src, att_dst, b_gat,
           W_gcn, b_gcn, bnA_g, bnA_b, bnM_g, bnM_b, mu, tau,
           W_prune, b_prune, W_rewire, b_rewire, W_pool, b_pool):
    astT = att_src.T
    adtT = att_dst.T
    bgat = b_gat.reshape(1, D)
    bgcn = b_gcn.reshape(1, D)
    bpool = b_pool.reshape(1, K_POOL)
    wg4 = jnp.stack([W_prune[:D], W_prune[D:2 * D],
                     W_rewire[:D], W_rewire[D:2 * D]], axis=1)
    wmpr = W_prune[2 * D].reshape(1, 1)
    bpr = jnp.asarray(b_prune, dtype=F32).reshape(1, 1)
    wmrw = W_rewire[2 * D].reshape(1, 1)
    brw = jnp.asarray(b_rewire, dtype=F32).reshape(1, 1)
    tau2 = jnp.asarray(tau, dtype=F32).reshape(1, 1)
    mu2 = jnp.asarray(mu, dtype=F32).reshape(1, 1)

    hx, a_s, a_d, xg = pl.pallas_call(
        _pre_kernel,
        out_shape=[_sds(N, D), _sds(N, HEADS), _sds(N, HEADS), _sds(N, D)],
    )(x, W_gat, W_gcn, astT, adtT)
    asT = a_s.T

    out_A, kth = pl.pallas_call(
        _gat_kth_kernel,
        grid=(GRID,),
        in_specs=[_blk((R, N)), _blk((R, N)), _full((N, D)),
                  _full((HEADS, N)), _blk((R, HEADS)), _full((1, D))],
        out_specs=[_blk((R, D)), _blk((R, 1))],
        out_shape=[_sds(N, D), _sds(N, 1)],
    )(A_in, A_motif, hx, asT, a_d, bgat)

    M_hat, dinv = pl.pallas_call(
        _mhat_kernel,
        grid=(GRID,),
        in_specs=[_blk((R, N)), _blk((R, 1)), _full((1, N))],
        out_specs=[_blk((R, N)), _blk((R, 1))],
        out_shape=[_sds(N, N), _sds(N, 1)],
    )(A_motif, kth, kth.reshape(1, N))

    out_M = pl.pallas_call(
        _gcn_kernel,
        grid=(GRID,),
        in_specs=[_blk((R, N)), _blk((R, 1)), _full((1, N)),
                  _full((N, D)), _full((1, D))],
        out_specs=_blk((R, D)),
        out_shape=_sds(N, D),
    )(M_hat, dinv, dinv.reshape(1, N), xg, bgcn)

    h, hp, gv = pl.pallas_call(
        _bn_kernel,
        out_shape=[_sds(N, D), _sds(N, K_POOL), _sds(N, 4)],
    )(out_A, out_M, bnA_g.reshape(1, D), bnA_b.reshape(1, D),
      bnM_g.reshape(1, D), bnM_b.reshape(1, D), mu2, W_pool, wg4)
    gvT = gv.T
    crdT = coords.T

    kth2 = pl.pallas_call(
        _kth2_kernel,
        grid=(GRID,),
        in_specs=[_blk((R, N)), _blk((R, N)), _blk((R, 4)), _full((4, N)),
                  _blk((R, 2)), _full((2, N)),
                  _full((1, 1)), _full((1, 1)), _full((1, 1))],
        out_specs=_blk((R, 1)),
        out_shape=_sds(N, 1),
    )(A_in, M_hat, gv, gvT, coords, crdT, wmrw, brw, tau2)

    A_new, dpi = pl.pallas_call(
        _anew_kernel,
        grid=(GRID,),
        in_specs=[_blk((R, N)), _blk((R, N)), _blk((R, 4)), _full((4, N)),
                  _blk((R, 2)), _full((2, N)),
                  _full((1, 1)), _full((1, 1)), _full((1, 1)), _full((1, 1)),
                  _full((1, 1)), _blk((R, 1)), _full((1, N))],
        out_specs=[_blk((R, N)), _blk((R, 1))],
        out_shape=[_sds(N, N), _sds(N, 1)],
    )(A_in, M_hat, gv, gvT, coords, crdT, wmpr, bpr, wmrw, brw, tau2,
      kth2, kth2.reshape(1, N))

    S = pl.pallas_call(
        _pool_kernel,
        grid=(GRID,),
        in_specs=[_blk((R, N)), _blk((R, 1)), _full((1, N)),
                  _full((N, K_POOL)), _full((1, K_POOL))],
        out_specs=_blk((R, K_POOL)),
        out_shape=_sds(N, K_POOL),
    )(A_new, dpi, dpi.reshape(1, N), hp, bpool)

    A_pool, x_pool = pl.pallas_call(
        _final_kernel,
        grid=(GRID,),
        in_specs=[_blk((R, N)), _full((N, K_POOL)), _full((N, D))],
        out_specs=[_full((K_POOL, K_POOL)), _full((K_POOL, D))],
        out_shape=[_sds(K_POOL, K_POOL), _sds(K_POOL, D)],
    )(A_new, S, h)

    return x_pool, A_pool, S


# speedup vs baseline: 10.0526x; 1.4012x over previous
import jax
import jax.numpy as jnp
from jax.experimental import pallas as pl
from jax.experimental.pallas import tpu as pltpu

N = 2048
IN_C = 128
H = 64
HEADS = 2
D = HEADS * H
K_POOL = 64
MOTIF_TOPK = 8
RADIUS = 2.0
TOPK_ADD = 2

R = 256
GRID = N // R
PHASES = 7
NEG = -1e30
F32 = jnp.float32


def _dot(a, b):
    return jax.lax.dot_general(a, b, (((1,), (0,)), ((), ())),
                               preferred_element_type=F32)


def _sigmoid(v):
    return 1.0 / (1.0 + jnp.exp(-v))


def _gate(l, tau_c):
    return jnp.clip(_sigmoid(l / tau_c) * 1.2 - 0.1, 0.0, 1.0)


def _eye_r():
    rr = jax.lax.broadcasted_iota(jnp.int32, (R, R), 0)
    cc = jax.lax.broadcasted_iota(jnp.int32, (R, R), 1)
    return rr == cc


def _t(v, eyeb):
    return jnp.sum(jnp.where(eyeb, v, 0.0), axis=0, keepdims=True)


def _mega_kernel(x_ref, ain_ref, am_ref, crd_ref, crdt_ref,
                 wgat_ref, wgcn_ref, bsrc_ref, bdst_ref, bgat_ref, bgcn_ref,
                 bnag_ref, bnab_ref, bnmg_ref, bnmb_ref, mu_ref,
                 wpool_ref, bpool_ref, wg4_ref,
                 wmpr_ref, bpr_ref, wmrw_ref, brw_ref, tau_ref,
                 s_out_ref, apool_ref, xpool_ref,
                 hx_scr, xg_scr, ast_scr, ad_scr, outa_scr, outm_scr,
                 h_scr, hp_scr, gv_scr, gvt_scr, s_scr,
                 kthc_scr, kthr_scr, dinvc_scr, dinvr_scr,
                 k2c_scr, k2r_scr, dpic_scr, dpir_scr,
                 w_scr):
    p = pl.program_id(0)
    i = pl.program_id(1)
    sl = pl.ds(i * R, R)

    @pl.when((p == 0) & (i == 0))
    def _prologue():
        eye = _eye_r()
        xx = x_ref[...]
        hx = _dot(xx, wgat_ref[...])
        hx_scr[...] = hx
        xg_scr[...] = _dot(xx, wgcn_ref[...])
        a_s = _dot(hx, bsrc_ref[...])
        ad_scr[...] = _dot(hx, bdst_ref[...])
        for k in range(GRID):
            for hd in range(HEADS):
                ast_scr[hd:hd + 1, k * R:(k + 1) * R] = _t(
                    a_s[k * R:(k + 1) * R, hd:hd + 1], eye)

    @pl.when(p == 0)
    def _gat_kth():
        eye = _eye_r()
        ain = ain_ref[...]
        rows = jax.lax.broadcasted_iota(jnp.int32, (R, N), 0) + i * R
        cols = jax.lax.broadcasted_iota(jnp.int32, (R, N), 1)
        mask = (ain > 0) | (rows == cols)
        outs = []
        for hd in range(HEADS):
            e = ad_scr[sl, hd:hd + 1] + ast_scr[hd:hd + 1, :]
            e = jnp.where(e >= 0, e, 0.2 * e)
            e = jnp.where(mask, e, NEG)
            m = jnp.max(e, axis=1, keepdims=True)
            pr = jnp.where(mask, jnp.exp(e - m), 0.0)
            s = jnp.sum(pr, axis=1, keepdims=True)
            outs.append(_dot(pr / s, hx_scr[:, hd * H:(hd + 1) * H]))
        outa_scr[sl, :] = jnp.concatenate(outs, axis=1) + bgat_ref[...]
        work = am_ref[...]
        m8 = jnp.max(work, axis=1, keepdims=True)
        for _ in range(MOTIF_TOPK - 1):
            work = jnp.where(work >= m8, NEG, work)
            m8 = jnp.max(work, axis=1, keepdims=True)
        kthc_scr[sl, :] = m8
        kthr_scr[:, sl] = _t(m8, eye)

    @pl.when(p == 1)
    def _mhat():
        eye = _eye_r()
        a = am_ref[...]
        keep = ((a >= kthc_scr[sl, :]) | (a >= kthr_scr[...])) & (a > 0)
        m = jnp.where(keep, a, 0.0)
        w_scr[sl, :] = m
        deg = jnp.sum(m, axis=1, keepdims=True)
        dinv = jnp.where(deg > 0,
                         jax.lax.rsqrt(jnp.where(deg > 0, deg, 1.0)), 0.0)
        dinvc_scr[sl, :] = dinv
        dinvr_scr[:, sl] = _t(dinv, eye)

    @pl.when(p == 2)
    def _gcn():
        t = w_scr[sl, :] * dinvr_scr[...]
        outm_scr[sl, :] = (dinvc_scr[sl, :] * _dot(t, xg_scr[...])
                           + bgcn_ref[...])

    @pl.when((p == 2) & (i == GRID - 1))
    def _bn():
        eye = _eye_r()

        def bn(v, g, b):
            m = jnp.mean(v, axis=0, keepdims=True)
            var = jnp.mean((v - m) ** 2, axis=0, keepdims=True)
            return (v - m) * jax.lax.rsqrt(var + 1e-5) * g + b

        def elu(v):
            return jnp.where(v > 0, v, jnp.exp(jnp.minimum(v, 0.0)) - 1.0)

        h_a = elu(bn(outa_scr[...], bnag_ref[...], bnab_ref[...]))
        h_m = elu(bn(outm_scr[...], bnmg_ref[...], bnmb_ref[...]))
        mu = mu_ref[...]
        softplus_mu = jnp.maximum(mu, 0.0) + jnp.log1p(jnp.exp(-jnp.abs(mu)))
        hh = h_a + softplus_mu * h_m
        h_scr[...] = hh
        hp_scr[...] = _dot(hh, wpool_ref[...])
        gv = _dot(hh, wg4_ref[...])
        gv_scr[...] = gv
        for k in range(GRID):
            for c in range(4):
                gvt_scr[c:c + 1, k * R:(k + 1) * R] = _t(
                    gv[k * R:(k + 1) * R, c:c + 1], eye)

    @pl.when(p == 3)
    def _scores():
        eye = _eye_r()
        rows = jax.lax.broadcasted_iota(jnp.int32, (R, N), 0) + i * R
        cols = jax.lax.broadcasted_iota(jnp.int32, (R, N), 1)
        upper = cols > rows
        ain = ain_ref[...]
        mh = w_scr[sl, :]
        gvb = gv_scr[sl, :]
        gvt = gvt_scr[...]
        tau_c = jnp.maximum(tau_ref[...], 0.1)
        dist = (jnp.abs(crd_ref[:, 0:1] - crdt_ref[0:1, :]) +
                jnp.abs(crd_ref[:, 1:2] - crdt_ref[1:2, :]))
        cand = (dist > 0) & (dist <= RADIUS) & (ain < 1e-6)
        l_rw = jnp.where(upper, gvb[:, 2:3] + gvt[3:4, :],
                         gvt[2:3, :] + gvb[:, 3:4])
        l_rw = l_rw + mh * wmrw_ref[...] + brw_ref[...]
        z = _gate(l_rw, tau_c)
        scores = jnp.where(cand, z, NEG)
        w_scr[sl, :] = scores
        m1 = jnp.max(scores, axis=1, keepdims=True)
        c = jnp.sum(jnp.where(scores == m1, 1.0, 0.0), axis=1, keepdims=True)
        m2 = jnp.max(jnp.where(scores < m1, scores, NEG), axis=1,
                     keepdims=True)
        k2 = jnp.where(c >= 2.0, m1, m2)
        k2c_scr[sl, :] = k2
        k2r_scr[:, sl] = _t(k2, eye)

    @pl.when(p == 4)
    def _anew():
        eye = _eye_r()
        rows = jax.lax.broadcasted_iota(jnp.int32, (R, N), 0) + i * R
        cols = jax.lax.broadcasted_iota(jnp.int32, (R, N), 1)
        upper = cols > rows
        ain = ain_ref[...]
        a = am_ref[...]
        keep = ((a >= kthc_scr[sl, :]) | (a >= kthr_scr[...])) & (a > 0)
        mh = jnp.where(keep, a, 0.0)
        tau_c = jnp.maximum(tau_ref[...], 0.1)
        gvb = gv_scr[sl, :]
        gvt = gvt_scr[...]
        l_pr = jnp.where(upper, gvb[:, 0:1] + gvt[1:2, :],
                         gvt[0:1, :] + gvb[:, 1:2])
        l_pr = l_pr + mh * wmpr_ref[...] + bpr_ref[...]
        sc = w_scr[sl, :]
        add = (sc >= 0.0) & ((sc >= k2c_scr[sl, :]) | (sc >= k2r_scr[...]))
        anew = ain * _gate(l_pr, tau_c) + jnp.where(add, sc, 0.0)
        w_scr[sl, :] = anew
        dp = jnp.sum(anew, axis=1, keepdims=True) + 1.0
        dpi = jax.lax.rsqrt(dp)
        dpic_scr[sl, :] = dpi
        dpir_scr[:, sl] = _t(dpi, eye)

    @pl.when(p == 5)
    def _pool():
        an = w_scr[sl, :]
        dpic = dpic_scr[sl, :]
        t = _dot(an * dpir_scr[...], hp_scr[...])
        logits = dpic * (t + dpic * hp_scr[sl, :]) + bpool_ref[...]
        m = jnp.max(logits, axis=1, keepdims=True)
        e = jnp.exp(logits - m)
        s_scr[sl, :] = e / jnp.sum(e, axis=1, keepdims=True)

    @pl.when(p == 6)
    def _final():
        an = w_scr[sl, :]
        s_blk = s_scr[sl, :]
        h_blk = h_scr[sl, :]
        y = _dot(an, s_scr[...])
        dn = (((0,), (0,)), ((), ()))
        ca = jax.lax.dot_general(s_blk, y, dn, preferred_element_type=F32)
        cx = jax.lax.dot_general(s_blk, h_blk, dn, preferred_element_type=F32)

        @pl.when(i == 0)
        def _():
            apool_ref[...] = ca
            xpool_ref[...] = cx

        @pl.when(i > 0)
        def _():
            apool_ref[...] = apool_ref[...] + ca
            xpool_ref[...] = xpool_ref[...] + cx

        @pl.when(i == GRID - 1)
        def _():
            s_out_ref[...] = s_scr[...]


def _vmem(*shape):
    return pltpu.VMEM(shape, F32)


def kernel(x, A_in, A_motif, coords, W_gat, att_src, att_dst, b_gat,
           W_gcn, b_gcn, bnA_g, bnA_b, bnM_g, bnM_b, mu, tau,
           W_prune, b_prune, W_rewire, b_rewire, W_pool, b_pool):
    zpad = jnp.zeros((H, 1), dtype=F32)
    bsrc = jnp.concatenate([
        jnp.concatenate([att_src[0][:, None], zpad], axis=1),
        jnp.concatenate([zpad, att_src[1][:, None]], axis=1)], axis=0)
    bdst = jnp.concatenate([
        jnp.concatenate([att_dst[0][:, None], zpad], axis=1),
        jnp.concatenate([zpad, att_dst[1][:, None]], axis=1)], axis=0)
    bgat = b_gat.reshape(1, D)
    bgcn = b_gcn.reshape(1, D)
    bpool = b_pool.reshape(1, K_POOL)
    wg4 = jnp.stack([W_prune[:D], W_prune[D:2 * D],
                     W_rewire[:D], W_rewire[D:2 * D]], axis=1)
    wmpr = W_prune[2 * D].reshape(1, 1)
    bpr = jnp.asarray(b_prune, dtype=F32).reshape(1, 1)
    wmrw = W_rewire[2 * D].reshape(1, 1)
    brw = jnp.asarray(b_rewire, dtype=F32).reshape(1, 1)
    tau2 = jnp.asarray(tau, dtype=F32).reshape(1, 1)
    mu2 = jnp.asarray(mu, dtype=F32).reshape(1, 1)
    crdt = coords.T

    def cmap(shape):
        return pl.BlockSpec(shape, lambda p, i: (0, 0))

    ain_spec = pl.BlockSpec(
        (R, N), lambda p, i: (jnp.where((p == 0) | (p == 3) | (p == 4), i, 0), 0))
    am_spec = pl.BlockSpec(
        (R, N), lambda p, i: (jnp.where((p <= 1) | (p == 4), i, 0), 0))
    crd_spec = pl.BlockSpec(
        (R, 2), lambda p, i: (jnp.where(p == 3, i, 0), 0))
    s_out_spec = pl.BlockSpec((N, K_POOL), lambda p, i: (0, 0))

    s_out, a_pool, x_pool = pl.pallas_call(
        _mega_kernel,
        grid=(PHASES, GRID),
        in_specs=[cmap((N, IN_C)), ain_spec, am_spec, crd_spec, cmap((2, N)),
                  cmap((IN_C, D)), cmap((IN_C, D)),
                  cmap((D, HEADS)), cmap((D, HEADS)),
                  cmap((1, D)), cmap((1, D)),
                  cmap((1, D)), cmap((1, D)), cmap((1, D)), cmap((1, D)),
                  cmap((1, 1)),
                  cmap((D, K_POOL)), cmap((1, K_POOL)), cmap((D, 4)),
                  cmap((1, 1)), cmap((1, 1)), cmap((1, 1)), cmap((1, 1)),
                  cmap((1, 1))],
        out_specs=[s_out_spec,
                   cmap((K_POOL, K_POOL)),
                   cmap((K_POOL, D))],
        out_shape=[jax.ShapeDtypeStruct((N, K_POOL), F32),
                   jax.ShapeDtypeStruct((K_POOL, K_POOL), F32),
                   jax.ShapeDtypeStruct((K_POOL, D), F32)],
        scratch_shapes=[
            _vmem(N, D), _vmem(N, D),
            _vmem(HEADS, N), _vmem(N, HEADS),
            _vmem(N, D), _vmem(N, D),
            _vmem(N, D), _vmem(N, K_POOL),
            _vmem(N, 4), _vmem(4, N),
            _vmem(N, K_POOL),
            _vmem(N, 1), _vmem(1, N),
            _vmem(N, 1), _vmem(1, N),
            _vmem(N, 1), _vmem(1, N),
            _vmem(N, 1), _vmem(1, N),
            _vmem(N, N),
        ],
    )(x, A_in, A_motif, coords, crdt,
      W_gat, W_gcn, bsrc, bdst, bgat, bgcn,
      bnA_g.reshape(1, D), bnA_b.reshape(1, D),
      bnM_g.reshape(1, D), bnM_b.reshape(1, D), mu2,
      W_pool, bpool, wg4, wmpr, bpr, wmrw, brw, tau2)

    return x_pool, a_pool, s_out


# speedup vs baseline: 10.2540x; 1.0200x over previous
import jax
import jax.numpy as jnp
from jax.experimental import pallas as pl
from jax.experimental.pallas import tpu as pltpu

N = 2048
IN_C = 128
H = 64
HEADS = 2
D = HEADS * H
K_POOL = 64
MOTIF_TOPK = 8
RADIUS = 2.0
TOPK_ADD = 2

R = 256
GRID = N // R
PHASES = 7
NEG = -1e30
F32 = jnp.float32


def _dot(a, b):
    return jax.lax.dot_general(a, b, (((1,), (0,)), ((), ())),
                               preferred_element_type=F32)


def _sigmoid(v):
    return 1.0 / (1.0 + jnp.exp(-v))


def _gate(l, tau_c):
    return jnp.clip(_sigmoid(l / tau_c) * 1.2 - 0.1, 0.0, 1.0)


def _eye_r():
    rr = jax.lax.broadcasted_iota(jnp.int32, (R, R), 0)
    cc = jax.lax.broadcasted_iota(jnp.int32, (R, R), 1)
    return rr == cc


def _t(v, eyeb):
    return jnp.sum(jnp.where(eyeb, v, 0.0), axis=0, keepdims=True)


def _mega_kernel(x_ref, ain_ref, am_ref, crd_ref, crdt_ref,
                 wgat_ref, wgcn_ref, bsrc_ref, bdst_ref, bgat_ref, bgcn_ref,
                 bnag_ref, bnab_ref, bnmg_ref, bnmb_ref, mu_ref,
                 wpool_ref, bpool_ref, wg4_ref,
                 wmpr_ref, bpr_ref, wmrw_ref, brw_ref, tau_ref,
                 s_out_ref, apool_ref, xpool_ref,
                 hx_scr, xg_scr, ast_scr, ad_scr, outa_scr, outm_scr,
                 h_scr, hp_scr, gv_scr, gvt_scr, s_scr,
                 kthc_scr, kthr_scr, dinvc_scr, dinvr_scr,
                 k2c_scr, k2r_scr, dpic_scr, dpir_scr,
                 ainb_scr, w_scr):
    p = pl.program_id(0)
    i = pl.program_id(1)
    sl = pl.ds(i * R, R)

    @pl.when((p == 0) & (i == 0))
    def _prologue():
        eye = _eye_r()
        xx = x_ref[...]
        hx = _dot(xx, wgat_ref[...])
        hx_scr[...] = hx
        xg_scr[...] = _dot(xx, wgcn_ref[...])
        a_s = _dot(hx, bsrc_ref[...])
        ad_scr[...] = _dot(hx, bdst_ref[...])
        for k in range(GRID):
            for hd in range(HEADS):
                ast_scr[hd:hd + 1, k * R:(k + 1) * R] = _t(
                    a_s[k * R:(k + 1) * R, hd:hd + 1], eye)

    @pl.when(p == 0)
    def _gat_kth():
        eye = _eye_r()
        ain = ain_ref[...]
        rows = jax.lax.broadcasted_iota(jnp.int32, (R, N), 0) + i * R
        cols = jax.lax.broadcasted_iota(jnp.int32, (R, N), 1)
        mask = (ain > 0) | (rows == cols)
        outs = []
        for hd in range(HEADS):
            e = ad_scr[sl, hd:hd + 1] + ast_scr[hd:hd + 1, :]
            e = jnp.where(e >= 0, e, 0.2 * e)
            e = jnp.where(mask, e, NEG)
            m = jnp.max(e, axis=1, keepdims=True)
            pr = jnp.where(mask, jnp.exp(e - m), 0.0)
            s = jnp.sum(pr, axis=1, keepdims=True)
            outs.append(_dot(pr / s, hx_scr[:, hd * H:(hd + 1) * H]))
        outa_scr[sl, :] = jnp.concatenate(outs, axis=1) + bgat_ref[...]
        ainb_scr[sl, :] = (ain > 0).astype(jnp.int8)
        work = am_ref[...]
        w_scr[sl, :] = work
        m8 = jnp.max(work, axis=1, keepdims=True)
        for _ in range(MOTIF_TOPK - 1):
            work = jnp.where(work >= m8, NEG, work)
            m8 = jnp.max(work, axis=1, keepdims=True)
        kthc_scr[sl, :] = m8
        kthr_scr[:, sl] = _t(m8, eye)

    @pl.when(p == 1)
    def _mhat():
        eye = _eye_r()
        a = w_scr[sl, :]
        keep = ((a >= kthc_scr[sl, :]) | (a >= kthr_scr[...])) & (a > 0)
        m = jnp.where(keep, a, 0.0)
        w_scr[sl, :] = m
        deg = jnp.sum(m, axis=1, keepdims=True)
        dinv = jnp.where(deg > 0,
                         jax.lax.rsqrt(jnp.where(deg > 0, deg, 1.0)), 0.0)
        dinvc_scr[sl, :] = dinv
        dinvr_scr[:, sl] = _t(dinv, eye)

    @pl.when(p == 2)
    def _gcn():
        t = w_scr[sl, :] * dinvr_scr[...]
        outm_scr[sl, :] = (dinvc_scr[sl, :] * _dot(t, xg_scr[...])
                           + bgcn_ref[...])

    @pl.when((p == 2) & (i == GRID - 1))
    def _bn():
        eye = _eye_r()

        def bn(v, g, b):
            m = jnp.mean(v, axis=0, keepdims=True)
            var = jnp.mean((v - m) ** 2, axis=0, keepdims=True)
            return (v - m) * jax.lax.rsqrt(var + 1e-5) * g + b

        def elu(v):
            return jnp.where(v > 0, v, jnp.exp(jnp.minimum(v, 0.0)) - 1.0)

        h_a = elu(bn(outa_scr[...], bnag_ref[...], bnab_ref[...]))
        h_m = elu(bn(outm_scr[...], bnmg_ref[...], bnmb_ref[...]))
        mu = mu_ref[...]
        softplus_mu = jnp.maximum(mu, 0.0) + jnp.log1p(jnp.exp(-jnp.abs(mu)))
        hh = h_a + softplus_mu * h_m
        h_scr[...] = hh
        hp_scr[...] = _dot(hh, wpool_ref[...])
        gv = _dot(hh, wg4_ref[...])
        gv_scr[...] = gv
        for k in range(GRID):
            for c in range(4):
                gvt_scr[c:c + 1, k * R:(k + 1) * R] = _t(
                    gv[k * R:(k + 1) * R, c:c + 1], eye)

    @pl.when(p == 3)
    def _scores():
        eye = _eye_r()
        rows = jax.lax.broadcasted_iota(jnp.int32, (R, N), 0) + i * R
        cols = jax.lax.broadcasted_iota(jnp.int32, (R, N), 1)
        upper = cols > rows
        ain = ainb_scr[sl, :].astype(F32)
        mh = w_scr[sl, :]
        gvb = gv_scr[sl, :]
        gvt = gvt_scr[...]
        tau_c = jnp.maximum(tau_ref[...], 0.1)
        dist = (jnp.abs(crd_ref[:, 0:1] - crdt_ref[0:1, :]) +
                jnp.abs(crd_ref[:, 1:2] - crdt_ref[1:2, :]))
        cand = (dist > 0) & (dist <= RADIUS) & (ain < 1e-6)
        l_rw = jnp.where(upper, gvb[:, 2:3] + gvt[3:4, :],
                         gvt[2:3, :] + gvb[:, 3:4])
        l_rw = l_rw + mh * wmrw_ref[...] + brw_ref[...]
        z = _gate(l_rw, tau_c)
        l_pr = jnp.where(upper, gvb[:, 0:1] + gvt[1:2, :],
                         gvt[0:1, :] + gvb[:, 1:2])
        l_pr = l_pr + mh * wmpr_ref[...] + bpr_ref[...]
        a_pruned = ain * _gate(l_pr, tau_c)
        w_scr[sl, :] = jnp.where(cand, 2.0 + z, a_pruned)
        enc = w_scr[sl, :]
        scores = jnp.where(cand, enc - 2.0, NEG)
        m1 = jnp.max(scores, axis=1, keepdims=True)
        c = jnp.sum(jnp.where(scores == m1, 1.0, 0.0), axis=1, keepdims=True)
        m2 = jnp.max(jnp.where(scores < m1, scores, NEG), axis=1,
                     keepdims=True)
        k2 = jnp.where(c >= 2.0, m1, m2)
        k2c_scr[sl, :] = k2
        k2r_scr[:, sl] = _t(k2, eye)

    @pl.when(p == 4)
    def _anew():
        eye = _eye_r()
        w = w_scr[sl, :]
        is_c = w >= 2.0
        sc = w - 2.0
        add = is_c & ((sc >= k2c_scr[sl, :]) | (sc >= k2r_scr[...]))
        anew = jnp.where(is_c, jnp.where(add, sc, 0.0), w)
        w_scr[sl, :] = anew
        dp = jnp.sum(anew, axis=1, keepdims=True) + 1.0
        dpi = jax.lax.rsqrt(dp)
        dpic_scr[sl, :] = dpi
        dpir_scr[:, sl] = _t(dpi, eye)

    @pl.when(p == 5)
    def _pool():
        an = w_scr[sl, :]
        dpic = dpic_scr[sl, :]
        t = _dot(an * dpir_scr[...], hp_scr[...])
        logits = dpic * (t + dpic * hp_scr[sl, :]) + bpool_ref[...]
        m = jnp.max(logits, axis=1, keepdims=True)
        e = jnp.exp(logits - m)
        s_scr[sl, :] = e / jnp.sum(e, axis=1, keepdims=True)

    @pl.when(p == 6)
    def _final():
        an = w_scr[sl, :]
        s_blk = s_scr[sl, :]
        h_blk = h_scr[sl, :]
        y = _dot(an, s_scr[...])
        dn = (((0,), (0,)), ((), ()))
        ca = jax.lax.dot_general(s_blk, y, dn, preferred_element_type=F32)
        cx = jax.lax.dot_general(s_blk, h_blk, dn, preferred_element_type=F32)

        @pl.when(i == 0)
        def _():
            apool_ref[...] = ca
            xpool_ref[...] = cx

        @pl.when(i > 0)
        def _():
            apool_ref[...] = apool_ref[...] + ca
            xpool_ref[...] = xpool_ref[...] + cx

        @pl.when(i == GRID - 1)
        def _():
            s_out_ref[...] = s_scr[...]


def _vmem(*shape):
    return pltpu.VMEM(shape, F32)


def kernel(x, A_in, A_motif, coords, W_gat, att_src, att_dst, b_gat,
           W_gcn, b_gcn, bnA_g, bnA_b, bnM_g, bnM_b, mu, tau,
           W_prune, b_prune, W_rewire, b_rewire, W_pool, b_pool):
    zpad = jnp.zeros((H, 1), dtype=F32)
    bsrc = jnp.concatenate([
        jnp.concatenate([att_src[0][:, None], zpad], axis=1),
        jnp.concatenate([zpad, att_src[1][:, None]], axis=1)], axis=0)
    bdst = jnp.concatenate([
        jnp.concatenate([att_dst[0][:, None], zpad], axis=1),
        jnp.concatenate([zpad, att_dst[1][:, None]], axis=1)], axis=0)
    bgat = b_gat.reshape(1, D)
    bgcn = b_gcn.reshape(1, D)
    bpool = b_pool.reshape(1, K_POOL)
    wg4 = jnp.stack([W_prune[:D], W_prune[D:2 * D],
                     W_rewire[:D], W_rewire[D:2 * D]], axis=1)
    wmpr = W_prune[2 * D].reshape(1, 1)
    bpr = jnp.asarray(b_prune, dtype=F32).reshape(1, 1)
    wmrw = W_rewire[2 * D].reshape(1, 1)
    brw = jnp.asarray(b_rewire, dtype=F32).reshape(1, 1)
    tau2 = jnp.asarray(tau, dtype=F32).reshape(1, 1)
    mu2 = jnp.asarray(mu, dtype=F32).reshape(1, 1)
    crdt = coords.T

    def cmap(shape):
        return pl.BlockSpec(shape, lambda p, i: (0, 0))

    ain_spec = pl.BlockSpec(
        (R, N), lambda p, i: (jnp.where(p == 0, i, 0), 0))
    am_spec = pl.BlockSpec(
        (R, N), lambda p, i: (jnp.where(p == 0, i, 0), 0))
    crd_spec = pl.BlockSpec(
        (R, 2), lambda p, i: (jnp.where(p == 3, i, 0), 0))
    s_out_spec = pl.BlockSpec((N, K_POOL), lambda p, i: (0, 0))

    s_out, a_pool, x_pool = pl.pallas_call(
        _mega_kernel,
        grid=(PHASES, GRID),
        in_specs=[cmap((N, IN_C)), ain_spec, am_spec, crd_spec, cmap((2, N)),
                  cmap((IN_C, D)), cmap((IN_C, D)),
                  cmap((D, HEADS)), cmap((D, HEADS)),
                  cmap((1, D)), cmap((1, D)),
                  cmap((1, D)), cmap((1, D)), cmap((1, D)), cmap((1, D)),
                  cmap((1, 1)),
                  cmap((D, K_POOL)), cmap((1, K_POOL)), cmap((D, 4)),
                  cmap((1, 1)), cmap((1, 1)), cmap((1, 1)), cmap((1, 1)),
                  cmap((1, 1))],
        out_specs=[s_out_spec,
                   cmap((K_POOL, K_POOL)),
                   cmap((K_POOL, D))],
        out_shape=[jax.ShapeDtypeStruct((N, K_POOL), F32),
                   jax.ShapeDtypeStruct((K_POOL, K_POOL), F32),
                   jax.ShapeDtypeStruct((K_POOL, D), F32)],
        scratch_shapes=[
            _vmem(N, D), _vmem(N, D),
            _vmem(HEADS, N), _vmem(N, HEADS),
            _vmem(N, D), _vmem(N, D),
            _vmem(N, D), _vmem(N, K_POOL),
            _vmem(N, 4), _vmem(4, N),
            _vmem(N, K_POOL),
            _vmem(N, 1), _vmem(1, N),
            _vmem(N, 1), _vmem(1, N),
            _vmem(N, 1), _vmem(1, N),
            _vmem(N, 1), _vmem(1, N),
            pltpu.VMEM((N, N), jnp.int8),
            _vmem(N, N),
        ],
    )(x, A_in, A_motif, coords, crdt,
      W_gat, W_gcn, bsrc, bdst, bgat, bgcn,
      bnA_g.reshape(1, D), bnA_b.reshape(1, D),
      bnM_g.reshape(1, D), bnM_b.reshape(1, D), mu2,
      W_pool, bpool, wg4, wmpr, bpr, wmrw, brw, tau2)

    return x_pool, a_pool, s_out
